# baseline (speedup 1.0000x reference)
import functools
import jax
import jax.numpy as jnp
import numpy as np
from jax import lax
from jax.experimental import pallas as pl
from jax.experimental.pallas import tpu as pltpu
from jax.experimental.pallas import tpu_sc as plsc

B = 16384
NC, NS = 2, 16
NW = NC * NS
CHUNK = B // NW
GPW = CHUNK // 16

SRC = (0, 0, 1, 1, 2, 2)
DST = (1, 2, 0, 2, 0, 1)
PAIRS = ((0, 1), (0, 2), (1, 2))
IN_EDGES = ((2, 1, 4, 2), (0, 0, 5, 2), (1, 0, 3, 1))

_F32 = jnp.float32

_WL0, _WR0, _WE0, _ATT0, _BL0, _BR0, _BIAS0 = 0, 224, 448, 496, 512, 528, 544
_WROWS = 560


def _rsqrt16(x):
    xi = lax.bitcast_convert_type(x, jnp.int32)
    yi = jnp.int32(0x5F3759DF) - lax.shift_right_arithmetic(xi, jnp.int32(1))
    y = lax.bitcast_convert_type(yi, _F32)
    for _ in range(3):
        y = y * (1.5 - 0.5 * x * y * y)
    return y


def _sc_gat_body(featG_hbm, wtab_hbm, poolT_hbm, featv, wv, xlsv, poolv):
    cid = lax.axis_index("c")
    sid = lax.axis_index("s")
    wid = sid * NC + cid
    pltpu.sync_copy(featG_hbm.at[pl.ds(wid * GPW, GPW)], featv)
    pltpu.sync_copy(wtab_hbm, wv)

    @plsc.parallel_loop(0, GPW, step=1)
    def gbody(g):
        geom = {}
        for (s, d) in PAIRS:
            cx = featv[g, 16 * d + 0, :] - featv[g, 16 * s + 0, :]
            cy = featv[g, 16 * d + 1, :] - featv[g, 16 * s + 1, :]
            d2 = cx * cx + cy * cy
            dist = d2 * _rsqrt16(d2)
            geom[(s, d)] = (cx, cy, dist)

        alphas = [jnp.zeros((16,), _F32) for _ in range(6)]
        for c in range(16):
            wl_c = [wv[_WL0 + k * 16 + c, :] for k in range(14)]
            wr_c = [wv[_WR0 + k * 16 + c, :] for k in range(14)]
            we_c = [wv[_WE0 + k * 16 + c, :] for k in range(3)]
            att_c = wv[_ATT0 + c, :]
            bl_c = wv[_BL0 + c, :]
            br_c = wv[_BR0 + c, :]

            xl_c, xr_c = [], []
            for j in range(3):
                al, ar = bl_c, br_c
                for k in range(14):
                    fk = featv[g, 16 * j + k, :]
                    al = al + fk * wl_c[k]
                    ar = ar + fk * wr_c[k]
                xl_c.append(al)
                xr_c.append(ar)

            ev = {}
            for (s, d) in PAIRS:
                cx, cy, dist = geom[(s, d)]
                u = cx * we_c[0] + cy * we_c[1]
                t = dist * we_c[2]
                ev[(s, d)] = t + u
                ev[(d, s)] = t - u

            for e in range(6):
                s, d = SRC[e], DST[e]
                m = xl_c[s] + xr_c[d] + ev[(s, d)]
                m = jnp.where(m > 0, m, 0.2 * m)
                alphas[e] = alphas[e] + m * att_c

            for j in range(3):
                xlsv[g, j, c, :] = xl_c[j]

        w = {}
        for dd in range(3):
            e1, s1, e2, s2 = IN_EDGES[dd]
            a1, a2 = alphas[e1], alphas[e2]
            amax = jnp.maximum(a1, a2)
            x1 = jnp.exp(a1 - amax)
            x2 = jnp.exp(a2 - amax)
            inv = 1.0 / (x1 + x2 + 1e-16)
            w[dd] = (x1 * inv, x2 * inv)

        for c in range(16):
            bias_c = wv[_BIAS0 + c, :]
            xj = [xlsv[g, j, c, :] for j in range(3)]
            pool_c = jnp.zeros((16,), _F32)
            for dd in range(3):
                _, s1, _, s2 = IN_EDGES[dd]
                w1, w2 = w[dd]
                o = w1 * xj[s1] + w2 * xj[s2] + bias_c
                pool_c = pool_c + jnp.maximum(o, 0.0)
            poolv[c, pl.ds(16 * g, 16)] = pool_c

    pltpu.sync_copy(poolv, poolT_hbm.at[:, pl.ds(wid * CHUNK, CHUNK)])


def _sc_gat(featG, wtab):
    mesh = plsc.VectorSubcoreMesh(core_axis_name="c", subcore_axis_name="s")
    return pl.kernel(
        _sc_gat_body,
        mesh=mesh,
        compiler_params=pltpu.CompilerParams(use_tc_tiling_on_sc=False),
        out_type=jax.ShapeDtypeStruct((16, B), _F32),
        scratch_types=[
            pltpu.VMEM((GPW, 48, 16), _F32),
            pltpu.VMEM((_WROWS, 16), _F32),
            pltpu.VMEM((GPW, 3, 16, 16), _F32),
            pltpu.VMEM((16, CHUNK), _F32),
        ],
    )(featG, wtab)


TILE_H = 2048


def _dot(a, b):
    return jax.lax.dot(a, b, preferred_element_type=_F32)


def _tc_head_body(poolT_ref, lmT_ref, WlT_ref, blb_ref, W1T_ref, b1_ref,
                  W2T_ref, b2_ref, outT_ref):
    WlT = WlT_ref[:, :]
    VT = WlT[:, 6:10] + WlT[:, 10:14]
    AT = jnp.concatenate([
        WlT[:, 0:1] - VT[:, 0:1] - VT[:, 2:3],
        WlT[:, 1:2] - VT[:, 1:2] - VT[:, 3:4],
        VT,
    ], axis=1)
    xlobj = _dot(AT, lmT_ref[:, :]) + blb_ref[:, :]
    objpool = 3.0 * jnp.maximum(xlobj, 0.0)
    h = jnp.concatenate([poolT_ref[:, :], objpool], axis=0)
    hid = jnp.maximum(_dot(W1T_ref[:, :], h) + b1_ref[:, :], 0.0)
    outT_ref[:, :] = _dot(W2T_ref[:, :], hid) + b2_ref[:, :]


def _tc_head(poolT, lmT, WlT, blb, W1T, b1, W2T, b2):
    grid = (B // TILE_H,)
    full = lambda shape: pl.BlockSpec(shape, lambda i: (0, 0))
    return pl.pallas_call(
        _tc_head_body,
        grid=grid,
        in_specs=[
            pl.BlockSpec((16, TILE_H), lambda i: (0, i)),
            pl.BlockSpec((6, TILE_H), lambda i: (0, i)),
            full((16, 14)),
            full((16, 1)),
            full((128, 32)),
            full((128, 1)),
            full((32, 128)),
            full((32, 1)),
        ],
        out_specs=pl.BlockSpec((32, TILE_H), lambda i: (0, i)),
        out_shape=jax.ShapeDtypeStruct((32, B), _F32),
    )(poolT, lmT, WlT, blb, W1T, b1, W2T, b2)


@jax.jit
def _run(featG, lmT, wtab, WlT, blb, W1T, b1, W2T, b2):
    poolT = _sc_gat(featG, wtab)
    return _tc_head(poolT, lmT, WlT, blb, W1T, b1, W2T, b2).T


def kernel(agent_pos, landmark_pos, agent_vel, other_pos, relative_landmark_pos,
           W_l, b_l, W_r, b_r, W_e, att, bias, W1, b1, W2, b2):
    b = agent_pos.shape[0]
    feat = jnp.concatenate(
        [agent_pos, agent_vel, relative_landmark_pos, other_pos,
         jnp.zeros((b, 3, 2), _F32)], axis=2).reshape(b // 16, 16, 48)
    featG = feat.transpose(0, 2, 1)
    lmT = landmark_pos.reshape(b, 6).T
    flat = jnp.concatenate([
        W_l.reshape(-1), W_r.reshape(-1), W_e.reshape(-1),
        att, b_l, b_r, bias])
    wtab = jnp.broadcast_to(flat[:, None], (_WROWS, 16))
    return _run(featG, lmT, wtab, W_l.T, (b_l + bias)[:, None],
                W1.T, b1[:, None], W2.T, b2[:, None])


# speedup vs baseline: 1.5296x; 1.5296x over previous
import functools
import jax
import jax.numpy as jnp
import numpy as np
from jax import lax
from jax.experimental import pallas as pl
from jax.experimental.pallas import tpu as pltpu
from jax.experimental.pallas import tpu_sc as plsc

B = 16384
NC, NS = 2, 16
NW = NC * NS
CHUNK = B // NW
GPW = CHUNK // 16

SRC = (0, 0, 1, 1, 2, 2)
DST = (1, 2, 0, 2, 0, 1)
PAIRS = ((0, 1), (0, 2), (1, 2))
IN_EDGES = ((2, 1, 4, 2), (0, 0, 5, 2), (1, 0, 3, 1))

_F32 = jnp.float32

_WL0, _WR0, _WE0, _ATT0, _BL0, _BR0, _BIAS0 = 0, 224, 448, 496, 512, 528, 544
_WROWS = 560


def _rsqrt16(x):
    xi = lax.bitcast_convert_type(x, jnp.int32)
    yi = jnp.int32(0x5F3759DF) - lax.shift_right_arithmetic(xi, jnp.int32(1))
    y = lax.bitcast_convert_type(yi, _F32)
    for _ in range(3):
        y = y * (1.5 - 0.5 * x * y * y)
    return y


def _sc_gat_body(featG_hbm, wtab_hbm, poolT_hbm, featv, wv, xlsv, poolv):
    cid = lax.axis_index("c")
    sid = lax.axis_index("s")
    wid = sid * NC + cid
    pltpu.sync_copy(featG_hbm.at[pl.ds(wid * GPW, GPW)], featv)
    pltpu.sync_copy(wtab_hbm, wv)

    def gbody(g, gcarry):
        geom = []
        for (s, d) in PAIRS:
            cx = featv[g, 16 * d + 0, :] - featv[g, 16 * s + 0, :]
            cy = featv[g, 16 * d + 1, :] - featv[g, 16 * s + 1, :]
            d2 = cx * cx + cy * cy
            dist = d2 * _rsqrt16(d2)
            geom.append((cx, cy, dist))

        def cbody(c, alphas):
            wl_c = [wv[_WL0 + k * 16 + c, :] for k in range(14)]
            wr_c = [wv[_WR0 + k * 16 + c, :] for k in range(14)]
            we_c = [wv[_WE0 + k * 16 + c, :] for k in range(3)]
            att_c = wv[_ATT0 + c, :]
            bl_c = wv[_BL0 + c, :]
            br_c = wv[_BR0 + c, :]

            xl_c, xr_c = [], []
            for j in range(3):
                al, ar = bl_c, br_c
                for k in range(14):
                    fk = featv[g, 16 * j + k, :]
                    al = al + fk * wl_c[k]
                    ar = ar + fk * wr_c[k]
                xl_c.append(al)
                xr_c.append(ar)

            ev = {}
            for p, (s, d) in enumerate(PAIRS):
                cx, cy, dist = geom[p]
                u = cx * we_c[0] + cy * we_c[1]
                t = dist * we_c[2]
                ev[(s, d)] = t + u
                ev[(d, s)] = t - u

            out = []
            for e in range(6):
                s, d = SRC[e], DST[e]
                m = xl_c[s] + xr_c[d] + ev[(s, d)]
                m = jnp.where(m > 0, m, 0.2 * m)
                out.append(alphas[e] + m * att_c)

            for j in range(3):
                xlsv[j, c, :] = xl_c[j]
            return tuple(out)

        alphas = lax.fori_loop(
            0, 16, cbody, tuple(jnp.zeros((16,), _F32) for _ in range(6)))

        w = []
        for dd in range(3):
            e1, s1, e2, s2 = IN_EDGES[dd]
            a1, a2 = alphas[e1], alphas[e2]
            amax = jnp.maximum(a1, a2)
            x1 = jnp.exp(a1 - amax)
            x2 = jnp.exp(a2 - amax)
            inv = 1.0 / (x1 + x2 + 1e-16)
            w.append((x1 * inv, x2 * inv))

        def c2body(c, z):
            bias_c = wv[_BIAS0 + c, :]
            xj = [xlsv[j, c, :] for j in range(3)]
            pool_c = jnp.zeros((16,), _F32)
            for dd in range(3):
                _, s1, _, s2 = IN_EDGES[dd]
                w1, w2 = w[dd]
                o = w1 * xj[s1] + w2 * xj[s2] + bias_c
                pool_c = pool_c + jnp.maximum(o, 0.0)
            poolv[c, pl.ds(16 * g, 16)] = pool_c
            return z

        lax.fori_loop(0, 16, c2body, 0)
        return gcarry

    lax.fori_loop(0, GPW, gbody, 0)

    pltpu.sync_copy(poolv, poolT_hbm.at[:, pl.ds(wid * CHUNK, CHUNK)])


def _sc_gat(featG, wtab):
    mesh = plsc.VectorSubcoreMesh(core_axis_name="c", subcore_axis_name="s")
    return pl.kernel(
        _sc_gat_body,
        mesh=mesh,
        compiler_params=pltpu.CompilerParams(use_tc_tiling_on_sc=False),
        out_type=jax.ShapeDtypeStruct((16, B), _F32),
        scratch_types=[
            pltpu.VMEM((GPW, 48, 16), _F32),
            pltpu.VMEM((_WROWS, 16), _F32),
            pltpu.VMEM((3, 16, 16), _F32),
            pltpu.VMEM((16, CHUNK), _F32),
        ],
    )(featG, wtab)


TILE_H = 2048


def _dot(a, b):
    return jax.lax.dot(a, b, preferred_element_type=_F32)


def _tc_head_body(poolT_ref, lmT_ref, WlT_ref, blb_ref, W1T_ref, b1_ref,
                  W2T_ref, b2_ref, outT_ref):
    WlT = WlT_ref[:, :]
    VT = WlT[:, 6:10] + WlT[:, 10:14]
    AT = jnp.concatenate([
        WlT[:, 0:1] - VT[:, 0:1] - VT[:, 2:3],
        WlT[:, 1:2] - VT[:, 1:2] - VT[:, 3:4],
        VT,
    ], axis=1)
    xlobj = _dot(AT, lmT_ref[:, :]) + blb_ref[:, :]
    objpool = 3.0 * jnp.maximum(xlobj, 0.0)
    h = jnp.concatenate([poolT_ref[:, :], objpool], axis=0)
    hid = jnp.maximum(_dot(W1T_ref[:, :], h) + b1_ref[:, :], 0.0)
    outT_ref[:, :] = _dot(W2T_ref[:, :], hid) + b2_ref[:, :]


def _tc_head(poolT, lmT, WlT, blb, W1T, b1, W2T, b2):
    grid = (B // TILE_H,)
    full = lambda shape: pl.BlockSpec(shape, lambda i: (0, 0))
    return pl.pallas_call(
        _tc_head_body,
        grid=grid,
        in_specs=[
            pl.BlockSpec((16, TILE_H), lambda i: (0, i)),
            pl.BlockSpec((6, TILE_H), lambda i: (0, i)),
            full((16, 14)),
            full((16, 1)),
            full((128, 32)),
            full((128, 1)),
            full((32, 128)),
            full((32, 1)),
        ],
        out_specs=pl.BlockSpec((32, TILE_H), lambda i: (0, i)),
        out_shape=jax.ShapeDtypeStruct((32, B), _F32),
    )(poolT, lmT, WlT, blb, W1T, b1, W2T, b2)


@jax.jit
def _run(featG, lmT, wtab, WlT, blb, W1T, b1, W2T, b2):
    poolT = _sc_gat(featG, wtab)
    return _tc_head(poolT, lmT, WlT, blb, W1T, b1, W2T, b2).T


def kernel(agent_pos, landmark_pos, agent_vel, other_pos, relative_landmark_pos,
           W_l, b_l, W_r, b_r, W_e, att, bias, W1, b1, W2, b2):
    b = agent_pos.shape[0]
    feat = jnp.concatenate(
        [agent_pos, agent_vel, relative_landmark_pos, other_pos,
         jnp.zeros((b, 3, 2), _F32)], axis=2).reshape(b // 16, 16, 48)
    featG = feat.transpose(0, 2, 1)
    lmT = landmark_pos.reshape(b, 6).T
    flat = jnp.concatenate([
        W_l.reshape(-1), W_r.reshape(-1), W_e.reshape(-1),
        att, b_l, b_r, bias])
    wtab = jnp.broadcast_to(flat[:, None], (_WROWS, 16))
    return _run(featG, lmT, wtab, W_l.T, (b_l + bias)[:, None],
                W1.T, b1[:, None], W2.T, b2[:, None])


# speedup vs baseline: 1.7697x; 1.1570x over previous
import functools
import jax
import jax.numpy as jnp
import numpy as np
from jax import lax
from jax.experimental import pallas as pl
from jax.experimental.pallas import tpu as pltpu
from jax.experimental.pallas import tpu_sc as plsc

B = 16384
S = 6144
NC, NS = 2, 16
NW = NC * NS
CHUNK = S // NW
GPW = CHUNK // 16
TILE = 512

SRC = (0, 0, 1, 1, 2, 2)
DST = (1, 2, 0, 2, 0, 1)
PAIRS = ((0, 1), (0, 2), (1, 2))
IN_EDGES = ((2, 1, 4, 2), (0, 0, 5, 2), (1, 0, 3, 1))

_F32 = jnp.float32

_WL0, _WR0, _WE0, _ATT0, _BL0, _BR0, _BIAS0 = 0, 224, 448, 496, 512, 528, 544
_WROWS = 560


def _dot(a, b):
    return jax.lax.dot(a, b, preferred_element_type=_F32)


def _prep_body(ap_ref, av_ref, rl_ref, op_ref, lm_ref, out_ref):
    apT = jnp.transpose(ap_ref[:, :])
    avT = jnp.transpose(av_ref[:, :])
    rlT = jnp.transpose(rl_ref[:, :])
    opT = jnp.transpose(op_ref[:, :])
    lmT = jnp.transpose(lm_ref[:, :])
    z = jnp.zeros((2, TILE), _F32)
    parts = []
    for j in range(3):
        parts += [apT[2 * j:2 * j + 2, :], avT[2 * j:2 * j + 2, :],
                  rlT[6 * j:6 * j + 6, :], opT[4 * j:4 * j + 4, :], z]
    parts.append(lmT)
    parts.append(z)
    out_ref[:, :] = jnp.concatenate(parts, axis=0)


def _prep(ap, av, rl, op, lm):
    grid = (B // TILE,)
    return pl.pallas_call(
        _prep_body,
        grid=grid,
        in_specs=[
            pl.BlockSpec((TILE, 6), lambda i: (i, 0)),
            pl.BlockSpec((TILE, 6), lambda i: (i, 0)),
            pl.BlockSpec((TILE, 18), lambda i: (i, 0)),
            pl.BlockSpec((TILE, 12), lambda i: (i, 0)),
            pl.BlockSpec((TILE, 6), lambda i: (i, 0)),
        ],
        out_specs=pl.BlockSpec((56, TILE), lambda i: (0, i)),
        out_shape=jax.ShapeDtypeStruct((56, B), _F32),
    )(ap, av, rl, op, lm)


def _rsqrt16(x):
    xi = lax.bitcast_convert_type(x, jnp.int32)
    yi = jnp.int32(0x5F3759DF) - lax.shift_right_arithmetic(xi, jnp.int32(1))
    y = lax.bitcast_convert_type(yi, _F32)
    for _ in range(3):
        y = y * (1.5 - 0.5 * x * y * y)
    return y


def _sc_gat_body(flt_hbm, wtab_hbm, pool_hbm, fv, wv, xlsv, poolv):
    cid = lax.axis_index("c")
    sid = lax.axis_index("s")
    wid = sid * NC + cid
    base = wid * CHUNK
    pltpu.sync_copy(flt_hbm.at[:, pl.ds(base, CHUNK)], fv)
    pltpu.sync_copy(wtab_hbm, wv)

    def gbody(g, gcarry):
        geom = []
        for (s, d) in PAIRS:
            cx = fv[16 * d + 0, pl.ds(16 * g, 16)] - fv[16 * s + 0, pl.ds(16 * g, 16)]
            cy = fv[16 * d + 1, pl.ds(16 * g, 16)] - fv[16 * s + 1, pl.ds(16 * g, 16)]
            d2 = cx * cx + cy * cy
            dist = d2 * _rsqrt16(d2)
            geom.append((cx, cy, dist))

        def cbody(c, alphas):
            wl_c = [wv[_WL0 + k * 16 + c, :] for k in range(14)]
            wr_c = [wv[_WR0 + k * 16 + c, :] for k in range(14)]
            we_c = [wv[_WE0 + k * 16 + c, :] for k in range(3)]
            att_c = wv[_ATT0 + c, :]
            bl_c = wv[_BL0 + c, :]
            br_c = wv[_BR0 + c, :]

            xl_c, xr_c = [], []
            for j in range(3):
                al, ar = bl_c, br_c
                for k in range(14):
                    fk = fv[16 * j + k, pl.ds(16 * g, 16)]
                    al = al + fk * wl_c[k]
                    ar = ar + fk * wr_c[k]
                xl_c.append(al)
                xr_c.append(ar)

            ev = {}
            for p, (s, d) in enumerate(PAIRS):
                cx, cy, dist = geom[p]
                u = cx * we_c[0] + cy * we_c[1]
                t = dist * we_c[2]
                ev[(s, d)] = t + u
                ev[(d, s)] = t - u

            out = []
            for e in range(6):
                s, d = SRC[e], DST[e]
                m = xl_c[s] + xr_c[d] + ev[(s, d)]
                m = jnp.where(m > 0, m, 0.2 * m)
                out.append(alphas[e] + m * att_c)

            for j in range(3):
                xlsv[j, c, :] = xl_c[j]
            return tuple(out)

        alphas = lax.fori_loop(
            0, 16, cbody, tuple(jnp.zeros((16,), _F32) for _ in range(6)))

        w = []
        for dd in range(3):
            e1, s1, e2, s2 = IN_EDGES[dd]
            a1, a2 = alphas[e1], alphas[e2]
            amax = jnp.maximum(a1, a2)
            x1 = jnp.exp(a1 - amax)
            x2 = jnp.exp(a2 - amax)
            inv = 1.0 / (x1 + x2 + 1e-16)
            w.append((x1 * inv, x2 * inv))

        def c2body(c, z):
            bias_c = wv[_BIAS0 + c, :]
            xj = [xlsv[j, c, :] for j in range(3)]
            pool_c = jnp.zeros((16,), _F32)
            for dd in range(3):
                _, s1, _, s2 = IN_EDGES[dd]
                w1, w2 = w[dd]
                o = w1 * xj[s1] + w2 * xj[s2] + bias_c
                pool_c = pool_c + jnp.maximum(o, 0.0)
            poolv[c, pl.ds(16 * g, 16)] = pool_c
            return z

        lax.fori_loop(0, 16, c2body, 0)
        return gcarry

    lax.fori_loop(0, GPW, gbody, 0)
    pltpu.sync_copy(poolv, pool_hbm.at[:, pl.ds(base, CHUNK)])


def _sc_gat(flt, wtab):
    mesh = plsc.VectorSubcoreMesh(core_axis_name="c", subcore_axis_name="s")
    return pl.kernel(
        _sc_gat_body,
        mesh=mesh,
        compiler_params=pltpu.CompilerParams(use_tc_tiling_on_sc=False),
        out_type=jax.ShapeDtypeStruct((16, S), _F32),
        scratch_types=[
            pltpu.VMEM((56, CHUNK), _F32),
            pltpu.VMEM((_WROWS, 16), _F32),
            pltpu.VMEM((3, 16, 16), _F32),
            pltpu.VMEM((16, CHUNK), _F32),
        ],
    )(flt, wtab)


def _obj_AT(WlT):
    VT = WlT[:, 6:10] + WlT[:, 10:14]
    return jnp.concatenate([
        WlT[:, 0:1] - VT[:, 0:1] - VT[:, 2:3],
        WlT[:, 1:2] - VT[:, 1:2] - VT[:, 3:4],
        VT,
    ], axis=1)


def _mlp(h, W1T_ref, b1_ref, W2T_ref, b2_ref):
    hid = jnp.maximum(_dot(W1T_ref[:, :], h) + b1_ref[:, :], 0.0)
    return _dot(W2T_ref[:, :], hid) + b2_ref[:, :]


def _tc_gat_body(flt_ref, WlrT_ref, WeT_ref, att_ref, blr_ref, bias_ref,
                 W1T_ref, b1_ref, W2T_ref, b2_ref, outT_ref):
    flt = flt_ref[:, :]
    WlrT = WlrT_ref[:, :]
    blr = blr_ref[:, :]
    bias = bias_ref[:, :]
    att = att_ref[:, :]
    WeT = WeT_ref[:, :]

    xl, xr, pos = [], [], []
    for j in range(3):
        fj = flt[16 * j:16 * j + 14, :]
        xlr = _dot(WlrT, fj) + blr
        xl.append(xlr[0:16, :])
        xr.append(xlr[16:32, :])
        pos.append(fj[0:2, :])

    alphas = []
    for e in range(6):
        s, d = SRC[e], DST[e]
        cx = pos[d][0:1, :] - pos[s][0:1, :]
        cy = pos[d][1:2, :] - pos[s][1:2, :]
        dist = jnp.sqrt(cx * cx + cy * cy)
        eT = WeT[:, 0:1] * cx + WeT[:, 1:2] * cy + WeT[:, 2:3] * dist
        m = xl[s] + xr[d] + eT
        m = jnp.where(m > 0, m, 0.2 * m)
        alphas.append(jnp.sum(m * att, axis=0, keepdims=True))

    pool = jnp.zeros((16, TILE), _F32)
    for d in range(3):
        e1, s1, e2, s2 = IN_EDGES[d]
        a1, a2 = alphas[e1], alphas[e2]
        amax = jnp.maximum(a1, a2)
        x1 = jnp.exp(a1 - amax)
        x2 = jnp.exp(a2 - amax)
        den = x1 + x2 + 1e-16
        o = (x1 / den) * xl[s1] + (x2 / den) * xl[s2] + bias
        pool = pool + jnp.maximum(o, 0.0)

    AT = _obj_AT(WlrT[0:16, :])
    xlobj = _dot(AT, flt[48:54, :]) + blr[0:16, :] + bias
    objpool = 3.0 * jnp.maximum(xlobj, 0.0)

    h = jnp.concatenate([pool, objpool], axis=0)
    outT_ref[:, :] = _mlp(h, W1T_ref, b1_ref, W2T_ref, b2_ref)


def _tc_gat(flt, WlrT, WeT, att2, blr, bias2, W1T, b1, W2T, b2):
    grid = ((B - S) // TILE,)
    off = S // TILE
    full = lambda shape: pl.BlockSpec(shape, lambda i: (0, 0))
    return pl.pallas_call(
        _tc_gat_body,
        grid=grid,
        in_specs=[
            pl.BlockSpec((56, TILE), lambda i: (0, i + off)),
            full((32, 14)),
            full((16, 3)),
            full((16, 1)),
            full((32, 1)),
            full((16, 1)),
            full((128, 32)),
            full((128, 1)),
            full((32, 128)),
            full((32, 1)),
        ],
        out_specs=pl.BlockSpec((32, TILE), lambda i: (0, i)),
        out_shape=jax.ShapeDtypeStruct((32, B - S), _F32),
    )(flt, WlrT, WeT, att2, blr, bias2, W1T, b1, W2T, b2)


def _sc_head_body(pool_ref, lmT_ref, WlT_ref, blb_ref, W1T_ref, b1_ref,
                  W2T_ref, b2_ref, outT_ref):
    AT = _obj_AT(WlT_ref[:, :])
    xlobj = _dot(AT, lmT_ref[0:6, :]) + blb_ref[:, :]
    objpool = 3.0 * jnp.maximum(xlobj, 0.0)
    h = jnp.concatenate([pool_ref[:, :], objpool], axis=0)
    outT_ref[:, :] = _mlp(h, W1T_ref, b1_ref, W2T_ref, b2_ref)


def _sc_head(pool, flt, WlT, blb, W1T, b1, W2T, b2):
    grid = (S // TILE,)
    full = lambda shape: pl.BlockSpec(shape, lambda i: (0, 0))
    return pl.pallas_call(
        _sc_head_body,
        grid=grid,
        in_specs=[
            pl.BlockSpec((16, TILE), lambda i: (0, i)),
            pl.BlockSpec((8, TILE), lambda i: (6, i)),
            full((16, 14)),
            full((16, 1)),
            full((128, 32)),
            full((128, 1)),
            full((32, 128)),
            full((32, 1)),
        ],
        out_specs=pl.BlockSpec((32, TILE), lambda i: (0, i)),
        out_shape=jax.ShapeDtypeStruct((32, S), _F32),
    )(pool, flt, WlT, blb, W1T, b1, W2T, b2)


@jax.jit
def _run(ap, av, rl, op, lm, wtab, WlrT, WeT, att2, blr, bias2, blb,
         W1T, b1, W2T, b2):
    flt = _prep(ap, av, rl, op, lm)
    pool_sc = _sc_gat(flt, wtab)
    out_tc = _tc_gat(flt, WlrT, WeT, att2, blr, bias2, W1T, b1, W2T, b2)
    out_sc = _sc_head(pool_sc, flt, WlrT[0:16, :], blb, W1T, b1, W2T, b2)
    return jnp.concatenate([out_sc, out_tc], axis=1).T


def kernel(agent_pos, landmark_pos, agent_vel, other_pos, relative_landmark_pos,
           W_l, b_l, W_r, b_r, W_e, att, bias, W1, b1, W2, b2):
    b = agent_pos.shape[0]
    ap = agent_pos.reshape(b, 6)
    av = agent_vel.reshape(b, 6)
    rl = relative_landmark_pos.reshape(b, 18)
    op = other_pos.reshape(b, 12)
    lm = landmark_pos.reshape(b, 6)
    flat = jnp.concatenate([
        W_l.reshape(-1), W_r.reshape(-1), W_e.reshape(-1),
        att, b_l, b_r, bias])
    wtab = jnp.broadcast_to(flat[:, None], (_WROWS, 16))
    WlrT = jnp.concatenate([W_l.T, W_r.T], axis=0)
    blr = jnp.concatenate([b_l, b_r])[:, None]
    return _run(ap, av, rl, op, lm, wtab, WlrT, W_e.T, att[:, None], blr,
                bias[:, None], (b_l + bias)[:, None],
                W1.T, b1[:, None], W2.T, b2[:, None])


# speedup vs baseline: 3.1605x; 1.7859x over previous
import functools
import jax
import jax.numpy as jnp
import numpy as np
from jax import lax
from jax.experimental import pallas as pl
from jax.experimental.pallas import tpu as pltpu
from jax.experimental.pallas import tpu_sc as plsc

B = 16384
S = 6144
NC, NS = 2, 16
NW = NC * NS
CHUNK = S // NW
GPW = CHUNK // 16
TILE = 512

SRC = (0, 0, 1, 1, 2, 2)
DST = (1, 2, 0, 2, 0, 1)
PAIRS = ((0, 1), (0, 2), (1, 2))
IN_EDGES = ((2, 1, 4, 2), (0, 0, 5, 2), (1, 0, 3, 1))

_F32 = jnp.float32

_WL0, _WR0, _WE0, _ATT0, _BL0, _BR0, _BIAS0 = 0, 224, 448, 496, 512, 528, 544
_WROWS = 560


def _dot(a, b):
    return jax.lax.dot(a, b, preferred_element_type=_F32)


def _rsqrt16(x):
    xi = lax.bitcast_convert_type(x, jnp.int32)
    yi = jnp.int32(0x5F3759DF) - lax.shift_right_arithmetic(xi, jnp.int32(1))
    y = lax.bitcast_convert_type(yi, _F32)
    for _ in range(3):
        y = y * (1.5 - 0.5 * x * y * y)
    return y


def _sc_gat_body(flt_hbm, wtab_hbm, pool_hbm, fv, wv, xlsv, poolv):
    cid = lax.axis_index("c")
    sid = lax.axis_index("s")
    wid = sid * NC + cid
    base = wid * CHUNK
    pltpu.sync_copy(flt_hbm.at[:, pl.ds(base, CHUNK)], fv)
    pltpu.sync_copy(wtab_hbm, wv)

    def gbody(g, gcarry):
        geom = []
        for (s, d) in PAIRS:
            cx = fv[14 * d + 0, pl.ds(16 * g, 16)] - fv[14 * s + 0, pl.ds(16 * g, 16)]
            cy = fv[14 * d + 1, pl.ds(16 * g, 16)] - fv[14 * s + 1, pl.ds(16 * g, 16)]
            d2 = cx * cx + cy * cy
            dist = d2 * _rsqrt16(d2)
            geom.append((cx, cy, dist))

        def cbody(c, alphas):
            wl_c = [wv[_WL0 + k * 16 + c, :] for k in range(14)]
            wr_c = [wv[_WR0 + k * 16 + c, :] for k in range(14)]
            we_c = [wv[_WE0 + k * 16 + c, :] for k in range(3)]
            att_c = wv[_ATT0 + c, :]
            bl_c = wv[_BL0 + c, :]
            br_c = wv[_BR0 + c, :]

            xl_c, xr_c = [], []
            for j in range(3):
                al, ar = bl_c, br_c
                for k in range(14):
                    fk = fv[14 * j + k, pl.ds(16 * g, 16)]
                    al = al + fk * wl_c[k]
                    ar = ar + fk * wr_c[k]
                xl_c.append(al)
                xr_c.append(ar)

            ev = {}
            for p, (s, d) in enumerate(PAIRS):
                cx, cy, dist = geom[p]
                u = cx * we_c[0] + cy * we_c[1]
                t = dist * we_c[2]
                ev[(s, d)] = t + u
                ev[(d, s)] = t - u

            out = []
            for e in range(6):
                s, d = SRC[e], DST[e]
                m = xl_c[s] + xr_c[d] + ev[(s, d)]
                m = jnp.where(m > 0, m, 0.2 * m)
                out.append(alphas[e] + m * att_c)

            for j in range(3):
                xlsv[j, c, :] = xl_c[j]
            return tuple(out)

        alphas = lax.fori_loop(
            0, 16, cbody, tuple(jnp.zeros((16,), _F32) for _ in range(6)))

        w = []
        for dd in range(3):
            e1, s1, e2, s2 = IN_EDGES[dd]
            a1, a2 = alphas[e1], alphas[e2]
            amax = jnp.maximum(a1, a2)
            x1 = jnp.exp(a1 - amax)
            x2 = jnp.exp(a2 - amax)
            inv = 1.0 / (x1 + x2 + 1e-16)
            w.append((x1 * inv, x2 * inv))

        def c2body(c, z):
            bias_c = wv[_BIAS0 + c, :]
            xj = [xlsv[j, c, :] for j in range(3)]
            pool_c = jnp.zeros((16,), _F32)
            for dd in range(3):
                _, s1, _, s2 = IN_EDGES[dd]
                w1, w2 = w[dd]
                o = w1 * xj[s1] + w2 * xj[s2] + bias_c
                pool_c = pool_c + jnp.maximum(o, 0.0)
            poolv[c, pl.ds(16 * g, 16)] = pool_c
            return z

        lax.fori_loop(0, 16, c2body, 0)
        return gcarry

    lax.fori_loop(0, GPW, gbody, 0)
    pltpu.sync_copy(poolv, pool_hbm.at[:, pl.ds(base, CHUNK)])


def _sc_gat(flt, wtab):
    mesh = plsc.VectorSubcoreMesh(core_axis_name="c", subcore_axis_name="s")
    return pl.kernel(
        _sc_gat_body,
        mesh=mesh,
        compiler_params=pltpu.CompilerParams(use_tc_tiling_on_sc=False),
        out_type=jax.ShapeDtypeStruct((16, S), _F32),
        scratch_types=[
            pltpu.VMEM((42, CHUNK), _F32),
            pltpu.VMEM((_WROWS, 16), _F32),
            pltpu.VMEM((3, 16, 16), _F32),
            pltpu.VMEM((16, CHUNK), _F32),
        ],
    )(flt, wtab)


def _obj_AT(WlT):
    VT = WlT[:, 6:10] + WlT[:, 10:14]
    return jnp.concatenate([
        WlT[:, 0:1] - VT[:, 0:1] - VT[:, 2:3],
        WlT[:, 1:2] - VT[:, 1:2] - VT[:, 3:4],
        VT,
    ], axis=1)


def _mlp(h, W1T_ref, b1_ref, W2T_ref, b2_ref):
    hid = jnp.maximum(_dot(W1T_ref[:, :], h) + b1_ref[:, :], 0.0)
    return _dot(W2T_ref[:, :], hid) + b2_ref[:, :]


def _tc_gat_body(flt_ref, lmT_ref, WlrT_ref, WeT_ref, att_ref, blr_ref, bias_ref,
                 W1T_ref, b1_ref, W2T_ref, b2_ref, outT_ref):
    flt = flt_ref[:, :]
    WlrT = WlrT_ref[:, :]
    blr = blr_ref[:, :]
    bias = bias_ref[:, :]
    att = att_ref[:, :]
    WeT = WeT_ref[:, :]

    xl, xr, pos = [], [], []
    for j in range(3):
        fj = flt[14 * j:14 * j + 14, :]
        xlr = _dot(WlrT, fj) + blr
        xl.append(xlr[0:16, :])
        xr.append(xlr[16:32, :])
        pos.append(fj[0:2, :])

    alphas = []
    for e in range(6):
        s, d = SRC[e], DST[e]
        cx = pos[d][0:1, :] - pos[s][0:1, :]
        cy = pos[d][1:2, :] - pos[s][1:2, :]
        dist = jnp.sqrt(cx * cx + cy * cy)
        eT = WeT[:, 0:1] * cx + WeT[:, 1:2] * cy + WeT[:, 2:3] * dist
        m = xl[s] + xr[d] + eT
        m = jnp.where(m > 0, m, 0.2 * m)
        alphas.append(jnp.sum(m * att, axis=0, keepdims=True))

    pool = jnp.zeros((16, TILE), _F32)
    for d in range(3):
        e1, s1, e2, s2 = IN_EDGES[d]
        a1, a2 = alphas[e1], alphas[e2]
        amax = jnp.maximum(a1, a2)
        x1 = jnp.exp(a1 - amax)
        x2 = jnp.exp(a2 - amax)
        den = x1 + x2 + 1e-16
        o = (x1 / den) * xl[s1] + (x2 / den) * xl[s2] + bias
        pool = pool + jnp.maximum(o, 0.0)

    AT = _obj_AT(WlrT[0:16, :])
    xlobj = _dot(AT, lmT_ref[:, :]) + blr[0:16, :] + bias
    objpool = 3.0 * jnp.maximum(xlobj, 0.0)

    h = jnp.concatenate([pool, objpool], axis=0)
    outT_ref[:, :] = _mlp(h, W1T_ref, b1_ref, W2T_ref, b2_ref)


def _tc_gat(flt, lmT, WlrT, WeT, att2, blr, bias2, W1T, b1, W2T, b2):
    grid = ((B - S) // TILE,)
    off = S // TILE
    full = lambda shape: pl.BlockSpec(shape, lambda i: (0, 0))
    return pl.pallas_call(
        _tc_gat_body,
        grid=grid,
        in_specs=[
            pl.BlockSpec((42, TILE), lambda i: (0, i + off)),
            pl.BlockSpec((6, TILE), lambda i: (0, i + off)),
            full((32, 14)),
            full((16, 3)),
            full((16, 1)),
            full((32, 1)),
            full((16, 1)),
            full((128, 32)),
            full((128, 1)),
            full((32, 128)),
            full((32, 1)),
        ],
        out_specs=pl.BlockSpec((32, TILE), lambda i: (0, i)),
        out_shape=jax.ShapeDtypeStruct((32, B - S), _F32),
    )(flt, lmT, WlrT, WeT, att2, blr, bias2, W1T, b1, W2T, b2)


def _sc_head_body(pool_ref, lmT_ref, WlT_ref, blb_ref, W1T_ref, b1_ref,
                  W2T_ref, b2_ref, outT_ref):
    AT = _obj_AT(WlT_ref[:, :])
    xlobj = _dot(AT, lmT_ref[:, :]) + blb_ref[:, :]
    objpool = 3.0 * jnp.maximum(xlobj, 0.0)
    h = jnp.concatenate([pool_ref[:, :], objpool], axis=0)
    outT_ref[:, :] = _mlp(h, W1T_ref, b1_ref, W2T_ref, b2_ref)


def _sc_head(pool, lmT, WlT, blb, W1T, b1, W2T, b2):
    grid = (S // TILE,)
    full = lambda shape: pl.BlockSpec(shape, lambda i: (0, 0))
    return pl.pallas_call(
        _sc_head_body,
        grid=grid,
        in_specs=[
            pl.BlockSpec((16, TILE), lambda i: (0, i)),
            pl.BlockSpec((6, TILE), lambda i: (0, i)),
            full((16, 14)),
            full((16, 1)),
            full((128, 32)),
            full((128, 1)),
            full((32, 128)),
            full((32, 1)),
        ],
        out_specs=pl.BlockSpec((32, TILE), lambda i: (0, i)),
        out_shape=jax.ShapeDtypeStruct((32, S), _F32),
    )(pool, lmT, WlT, blb, W1T, b1, W2T, b2)


@jax.jit
def _run(featT, lmT, wtab, WlrT, WeT, att2, blr, bias2, blb,
         W1T, b1, W2T, b2):
    pool_sc = _sc_gat(featT, wtab)
    out_tc = _tc_gat(featT, lmT, WlrT, WeT, att2, blr, bias2, W1T, b1, W2T, b2)
    out_sc = _sc_head(pool_sc, lmT, WlrT[0:16, :], blb, W1T, b1, W2T, b2)
    return jnp.concatenate([out_sc, out_tc], axis=1).T


def kernel(agent_pos, landmark_pos, agent_vel, other_pos, relative_landmark_pos,
           W_l, b_l, W_r, b_r, W_e, att, bias, W1, b1, W2, b2):
    b = agent_pos.shape[0]
    feat = jnp.concatenate(
        [agent_pos, agent_vel, relative_landmark_pos, other_pos], axis=2)
    featT = feat.reshape(b, 42).T
    lmT = landmark_pos.reshape(b, 6).T
    flat = jnp.concatenate([
        W_l.reshape(-1), W_r.reshape(-1), W_e.reshape(-1),
        att, b_l, b_r, bias])
    wtab = jnp.broadcast_to(flat[:, None], (_WROWS, 16))
    WlrT = jnp.concatenate([W_l.T, W_r.T], axis=0)
    blr = jnp.concatenate([b_l, b_r])[:, None]
    return _run(featT, lmT, wtab, WlrT, W_e.T, att[:, None], blr,
                bias[:, None], (b_l + bias)[:, None],
                W1.T, b1[:, None], W2.T, b2[:, None])


# speedup vs baseline: 3.7433x; 1.1844x over previous
import functools
import jax
import jax.numpy as jnp
import numpy as np
from jax import lax
from jax.experimental import pallas as pl
from jax.experimental.pallas import tpu as pltpu
from jax.experimental.pallas import tpu_sc as plsc

B = 16384
S = 8192
NC, NS = 2, 16
NW = NC * NS
CHUNK = S // NW
GPW = CHUNK // 16
TILE = 512

SRC = (0, 0, 1, 1, 2, 2)
DST = (1, 2, 0, 2, 0, 1)
PAIRS = ((0, 1), (0, 2), (1, 2))
IN_EDGES = ((2, 1, 4, 2), (0, 0, 5, 2), (1, 0, 3, 1))

_F32 = jnp.float32

_WL0, _WR0, _WE0, _ATT0, _BL0, _BR0, _BIAS0 = 0, 224, 448, 496, 512, 528, 544
_WROWS = 560


def _dot(a, b):
    return jax.lax.dot(a, b, preferred_element_type=_F32)


def _rsqrt16(x):
    xi = lax.bitcast_convert_type(x, jnp.int32)
    yi = jnp.int32(0x5F3759DF) - lax.shift_right_arithmetic(xi, jnp.int32(1))
    y = lax.bitcast_convert_type(yi, _F32)
    for _ in range(3):
        y = y * (1.5 - 0.5 * x * y * y)
    return y


def _sc_gat_body(flt_hbm, wtab_hbm, pool_hbm, fv, wv, xlsv, poolv):
    cid = lax.axis_index("c")
    sid = lax.axis_index("s")
    wid = sid * NC + cid
    base = wid * CHUNK
    pltpu.sync_copy(flt_hbm.at[:, pl.ds(base, CHUNK)], fv)
    pltpu.sync_copy(wtab_hbm, wv)

    def gbody(g, gcarry):
        geom = []
        for (s, d) in PAIRS:
            cx = fv[14 * d + 0, pl.ds(16 * g, 16)] - fv[14 * s + 0, pl.ds(16 * g, 16)]
            cy = fv[14 * d + 1, pl.ds(16 * g, 16)] - fv[14 * s + 1, pl.ds(16 * g, 16)]
            d2 = cx * cx + cy * cy
            dist = d2 * _rsqrt16(d2)
            geom.append((cx, cy, dist))

        def cbody(c, alphas):
            wl_c = [wv[_WL0 + k * 16 + c, :] for k in range(14)]
            wr_c = [wv[_WR0 + k * 16 + c, :] for k in range(14)]
            we_c = [wv[_WE0 + k * 16 + c, :] for k in range(3)]
            att_c = wv[_ATT0 + c, :]
            bl_c = wv[_BL0 + c, :]
            br_c = wv[_BR0 + c, :]

            xl_c, xr_c = [], []
            for j in range(3):
                al, ar = bl_c, br_c
                for k in range(14):
                    fk = fv[14 * j + k, pl.ds(16 * g, 16)]
                    al = al + fk * wl_c[k]
                    ar = ar + fk * wr_c[k]
                xl_c.append(al)
                xr_c.append(ar)

            ev = {}
            for p, (s, d) in enumerate(PAIRS):
                cx, cy, dist = geom[p]
                u = cx * we_c[0] + cy * we_c[1]
                t = dist * we_c[2]
                ev[(s, d)] = t + u
                ev[(d, s)] = t - u

            out = []
            for e in range(6):
                s, d = SRC[e], DST[e]
                m = xl_c[s] + xr_c[d] + ev[(s, d)]
                m = jnp.where(m > 0, m, 0.2 * m)
                out.append(alphas[e] + m * att_c)

            for j in range(3):
                xlsv[j, c, :] = xl_c[j]
            return tuple(out)

        alphas = lax.fori_loop(
            0, 16, cbody, tuple(jnp.zeros((16,), _F32) for _ in range(6)))

        w = []
        for dd in range(3):
            e1, s1, e2, s2 = IN_EDGES[dd]
            a1, a2 = alphas[e1], alphas[e2]
            amax = jnp.maximum(a1, a2)
            x1 = jnp.exp(a1 - amax)
            x2 = jnp.exp(a2 - amax)
            inv = 1.0 / (x1 + x2 + 1e-16)
            w.append((x1 * inv, x2 * inv))

        def c2body(c, z):
            bias_c = wv[_BIAS0 + c, :]
            xj = [xlsv[j, c, :] for j in range(3)]
            pool_c = jnp.zeros((16,), _F32)
            for dd in range(3):
                _, s1, _, s2 = IN_EDGES[dd]
                w1, w2 = w[dd]
                o = w1 * xj[s1] + w2 * xj[s2] + bias_c
                pool_c = pool_c + jnp.maximum(o, 0.0)
            poolv[c, pl.ds(16 * g, 16)] = pool_c
            return z

        lax.fori_loop(0, 16, c2body, 0)
        return gcarry

    lax.fori_loop(0, GPW, gbody, 0)
    pltpu.sync_copy(poolv, pool_hbm.at[:, pl.ds(base, CHUNK)])


def _sc_gat(flt, wtab):
    mesh = plsc.VectorSubcoreMesh(core_axis_name="c", subcore_axis_name="s")
    return pl.kernel(
        _sc_gat_body,
        mesh=mesh,
        compiler_params=pltpu.CompilerParams(use_tc_tiling_on_sc=False),
        out_type=jax.ShapeDtypeStruct((16, S), _F32),
        scratch_types=[
            pltpu.VMEM((42, CHUNK), _F32),
            pltpu.VMEM((_WROWS, 16), _F32),
            pltpu.VMEM((3, 16, 16), _F32),
            pltpu.VMEM((16, CHUNK), _F32),
        ],
    )(flt, wtab)


def _obj_AT(WlT):
    VT = WlT[:, 6:10] + WlT[:, 10:14]
    return jnp.concatenate([
        WlT[:, 0:1] - VT[:, 0:1] - VT[:, 2:3],
        WlT[:, 1:2] - VT[:, 1:2] - VT[:, 3:4],
        VT,
    ], axis=1)


def _mlp(h, W1T_ref, b1_ref, W2T_ref, b2_ref):
    hid = jnp.maximum(_dot(W1T_ref[:, :], h) + b1_ref[:, :], 0.0)
    return _dot(W2T_ref[:, :], hid) + b2_ref[:, :]


def _tc_gat_body(flt_ref, lmT_ref, WlrT_ref, WeT_ref, att_ref, blr_ref, bias_ref,
                 W1T_ref, b1_ref, W2T_ref, b2_ref, outT_ref):
    flt = flt_ref[:, :]
    WlrT = WlrT_ref[:, :]
    blr = blr_ref[:, :]
    bias = bias_ref[:, :]
    att = att_ref[:, :]
    WeT = WeT_ref[:, :]

    xl, xr, pos = [], [], []
    for j in range(3):
        fj = flt[14 * j:14 * j + 14, :]
        xlr = _dot(WlrT, fj) + blr
        xl.append(xlr[0:16, :])
        xr.append(xlr[16:32, :])
        pos.append(fj[0:2, :])

    alphas = []
    for e in range(6):
        s, d = SRC[e], DST[e]
        cx = pos[d][0:1, :] - pos[s][0:1, :]
        cy = pos[d][1:2, :] - pos[s][1:2, :]
        dist = jnp.sqrt(cx * cx + cy * cy)
        eT = WeT[:, 0:1] * cx + WeT[:, 1:2] * cy + WeT[:, 2:3] * dist
        m = xl[s] + xr[d] + eT
        m = jnp.where(m > 0, m, 0.2 * m)
        alphas.append(jnp.sum(m * att, axis=0, keepdims=True))

    pool = jnp.zeros((16, TILE), _F32)
    for d in range(3):
        e1, s1, e2, s2 = IN_EDGES[d]
        a1, a2 = alphas[e1], alphas[e2]
        amax = jnp.maximum(a1, a2)
        x1 = jnp.exp(a1 - amax)
        x2 = jnp.exp(a2 - amax)
        den = x1 + x2 + 1e-16
        o = (x1 / den) * xl[s1] + (x2 / den) * xl[s2] + bias
        pool = pool + jnp.maximum(o, 0.0)

    AT = _obj_AT(WlrT[0:16, :])
    xlobj = _dot(AT, lmT_ref[:, :]) + blr[0:16, :] + bias
    objpool = 3.0 * jnp.maximum(xlobj, 0.0)

    h = jnp.concatenate([pool, objpool], axis=0)
    outT_ref[:, :] = _mlp(h, W1T_ref, b1_ref, W2T_ref, b2_ref)


def _tc_gat(flt, lmT, WlrT, WeT, att2, blr, bias2, W1T, b1, W2T, b2):
    grid = ((B - S) // TILE,)
    full = lambda shape: pl.BlockSpec(shape, lambda i: (0, 0))
    return pl.pallas_call(
        _tc_gat_body,
        grid=grid,
        in_specs=[
            pl.BlockSpec((42, TILE), lambda i: (0, i)),
            pl.BlockSpec((6, TILE), lambda i: (0, i)),
            full((32, 14)),
            full((16, 3)),
            full((16, 1)),
            full((32, 1)),
            full((16, 1)),
            full((128, 32)),
            full((128, 1)),
            full((32, 128)),
            full((32, 1)),
        ],
        out_specs=pl.BlockSpec((32, TILE), lambda i: (0, i)),
        out_shape=jax.ShapeDtypeStruct((32, B - S), _F32),
    )(flt, lmT, WlrT, WeT, att2, blr, bias2, W1T, b1, W2T, b2)


def _sc_head_body(pool_ref, lmT_ref, WlT_ref, blb_ref, W1T_ref, b1_ref,
                  W2T_ref, b2_ref, outT_ref):
    AT = _obj_AT(WlT_ref[:, :])
    xlobj = _dot(AT, lmT_ref[:, :]) + blb_ref[:, :]
    objpool = 3.0 * jnp.maximum(xlobj, 0.0)
    h = jnp.concatenate([pool_ref[:, :], objpool], axis=0)
    outT_ref[:, :] = _mlp(h, W1T_ref, b1_ref, W2T_ref, b2_ref)


TILE_HD = 2048


def _sc_head(pool, lmT, WlT, blb, W1T, b1, W2T, b2):
    grid = (S // TILE_HD,)
    full = lambda shape: pl.BlockSpec(shape, lambda i: (0, 0))
    return pl.pallas_call(
        _sc_head_body,
        grid=grid,
        in_specs=[
            pl.BlockSpec((16, TILE_HD), lambda i: (0, i)),
            pl.BlockSpec((6, TILE_HD), lambda i: (0, i)),
            full((16, 14)),
            full((16, 1)),
            full((128, 32)),
            full((128, 1)),
            full((32, 128)),
            full((32, 1)),
        ],
        out_specs=pl.BlockSpec((32, TILE_HD), lambda i: (0, i)),
        out_shape=jax.ShapeDtypeStruct((32, S), _F32),
    )(pool, lmT, WlT, blb, W1T, b1, W2T, b2)


@jax.jit
def _run(featT_sc, lmT_sc, featT_tc, lmT_tc, wtab, WlrT, WeT, att2, blr,
         bias2, blb, W1T, b1, W2T, b2):
    pool_sc = _sc_gat(featT_sc, wtab)
    out_tc = _tc_gat(featT_tc, lmT_tc, WlrT, WeT, att2, blr, bias2,
                     W1T, b1, W2T, b2)
    out_sc = _sc_head(pool_sc, lmT_sc, WlrT[0:16, :], blb, W1T, b1, W2T, b2)
    return jnp.concatenate([out_sc, out_tc], axis=1).T


def kernel(agent_pos, landmark_pos, agent_vel, other_pos, relative_landmark_pos,
           W_l, b_l, W_r, b_r, W_e, att, bias, W1, b1, W2, b2):
    b = agent_pos.shape[0]
    feat_sc = jnp.concatenate(
        [agent_pos[:S], agent_vel[:S], relative_landmark_pos[:S],
         other_pos[:S]], axis=2)
    feat_tc = jnp.concatenate(
        [agent_pos[S:], agent_vel[S:], relative_landmark_pos[S:],
         other_pos[S:]], axis=2)
    featT_sc = feat_sc.reshape(S, 42).T
    featT_tc = feat_tc.reshape(b - S, 42).T
    lmT_sc = landmark_pos[:S].reshape(S, 6).T
    lmT_tc = landmark_pos[S:].reshape(b - S, 6).T
    flat = jnp.concatenate([
        W_l.reshape(-1), W_r.reshape(-1), W_e.reshape(-1),
        att, b_l, b_r, bias])
    wtab = jnp.broadcast_to(flat[:, None], (_WROWS, 16))
    WlrT = jnp.concatenate([W_l.T, W_r.T], axis=0)
    blr = jnp.concatenate([b_l, b_r])[:, None]
    return _run(featT_sc, lmT_sc, featT_tc, lmT_tc, wtab, WlrT, W_e.T,
                att[:, None], blr, bias[:, None], (b_l + bias)[:, None],
                W1.T, b1[:, None], W2.T, b2[:, None])


# speedup vs baseline: 3.7540x; 1.0029x over previous
import jax
import jax.numpy as jnp
from jax import lax
from jax.experimental import pallas as pl
from jax.experimental.pallas import tpu as pltpu
from jax.experimental.pallas import tpu_sc as plsc

B = 16384
S = 8192
NC, NS = 2, 16
NW = NC * NS
CHUNK = S // NW
GPW = CHUNK // 16
TILE = 512

SRC = (0, 0, 1, 1, 2, 2)
DST = (1, 2, 0, 2, 0, 1)
PAIRS = ((0, 1), (0, 2), (1, 2))
IN_EDGES = ((2, 1, 4, 2), (0, 0, 5, 2), (1, 0, 3, 1))

_F32 = jnp.float32

_WL0, _WR0, _WE0, _ATT0, _BL0, _BR0, _BIAS0 = 0, 224, 448, 496, 512, 528, 544
_WROWS = 560


def _dot(a, b):
    return jax.lax.dot(a, b, preferred_element_type=_F32)


def _rsqrt16(x):
    xi = lax.bitcast_convert_type(x, jnp.int32)
    yi = jnp.int32(0x5F3759DF) - lax.shift_right_arithmetic(xi, jnp.int32(1))
    y = lax.bitcast_convert_type(yi, _F32)
    for _ in range(3):
        y = y * (1.5 - 0.5 * x * y * y)
    return y


def _sc_gat_body(flt_hbm, wtab_hbm, pool_hbm, fv, wv, xlsv, poolv):
    cid = lax.axis_index("c")
    sid = lax.axis_index("s")
    wid = sid * NC + cid
    base = wid * CHUNK
    pltpu.sync_copy(flt_hbm.at[:, pl.ds(base, CHUNK)], fv)
    pltpu.sync_copy(wtab_hbm, wv)

    def gbody(g, gcarry):
        geom = []
        for (s, d) in PAIRS:
            cx = fv[14 * d + 0, pl.ds(16 * g, 16)] - fv[14 * s + 0, pl.ds(16 * g, 16)]
            cy = fv[14 * d + 1, pl.ds(16 * g, 16)] - fv[14 * s + 1, pl.ds(16 * g, 16)]
            d2 = cx * cx + cy * cy
            dist = d2 * _rsqrt16(d2)
            geom.append((cx, cy, dist))

        def cbody(c, alphas):
            wl_c = [wv[_WL0 + k * 16 + c, :] for k in range(14)]
            wr_c = [wv[_WR0 + k * 16 + c, :] for k in range(14)]
            we_c = [wv[_WE0 + k * 16 + c, :] for k in range(3)]
            att_c = wv[_ATT0 + c, :]
            bl_c = wv[_BL0 + c, :]
            br_c = wv[_BR0 + c, :]

            xl_c, xr_c = [], []
            for j in range(3):
                al, ar = bl_c, br_c
                for k in range(14):
                    fk = fv[14 * j + k, pl.ds(16 * g, 16)]
                    al = al + fk * wl_c[k]
                    ar = ar + fk * wr_c[k]
                xl_c.append(al)
                xr_c.append(ar)

            ev = {}
            for p, (s, d) in enumerate(PAIRS):
                cx, cy, dist = geom[p]
                u = cx * we_c[0] + cy * we_c[1]
                t = dist * we_c[2]
                ev[(s, d)] = t + u
                ev[(d, s)] = t - u

            out = []
            for e in range(6):
                s, d = SRC[e], DST[e]
                m = xl_c[s] + xr_c[d] + ev[(s, d)]
                m = jnp.where(m > 0, m, 0.2 * m)
                out.append(alphas[e] + m * att_c)

            for j in range(3):
                xlsv[j, c, :] = xl_c[j]
            return tuple(out)

        alphas = lax.fori_loop(
            0, 16, cbody, tuple(jnp.zeros((16,), _F32) for _ in range(6)))

        w = []
        for dd in range(3):
            e1, s1, e2, s2 = IN_EDGES[dd]
            a1, a2 = alphas[e1], alphas[e2]
            amax = jnp.maximum(a1, a2)
            x1 = jnp.exp(a1 - amax)
            x2 = jnp.exp(a2 - amax)
            inv = 1.0 / (x1 + x2 + 1e-16)
            w.append((x1 * inv, x2 * inv))

        def c2body(c, z):
            bias_c = wv[_BIAS0 + c, :]
            xj = [xlsv[j, c, :] for j in range(3)]
            pool_c = jnp.zeros((16,), _F32)
            for dd in range(3):
                _, s1, _, s2 = IN_EDGES[dd]
                w1, w2 = w[dd]
                o = w1 * xj[s1] + w2 * xj[s2] + bias_c
                pool_c = pool_c + jnp.maximum(o, 0.0)
            poolv[c, pl.ds(16 * g, 16)] = pool_c
            return z

        lax.fori_loop(0, 16, c2body, 0)
        return gcarry

    lax.fori_loop(0, GPW, gbody, 0)
    pltpu.sync_copy(poolv, pool_hbm.at[:, pl.ds(base, CHUNK)])


def _sc_gat(flt, wtab):
    mesh = plsc.VectorSubcoreMesh(core_axis_name="c", subcore_axis_name="s")
    return pl.kernel(
        _sc_gat_body,
        mesh=mesh,
        compiler_params=pltpu.CompilerParams(use_tc_tiling_on_sc=False),
        out_type=jax.ShapeDtypeStruct((16, S), _F32),
        scratch_types=[
            pltpu.VMEM((42, CHUNK), _F32),
            pltpu.VMEM((_WROWS, 16), _F32),
            pltpu.VMEM((3, 16, 16), _F32),
            pltpu.VMEM((16, CHUNK), _F32),
        ],
    )(flt, wtab)


def _obj_AT(WlT):
    VT = WlT[:, 6:10] + WlT[:, 10:14]
    return jnp.concatenate([
        WlT[:, 0:1] - VT[:, 0:1] - VT[:, 2:3],
        WlT[:, 1:2] - VT[:, 1:2] - VT[:, 3:4],
        VT,
    ], axis=1)


def _mlp(h, W1T_ref, b1_ref, W2T_ref, b2_ref):
    hid = jnp.maximum(_dot(W1T_ref[:, :], h) + b1_ref[:, :], 0.0)
    return _dot(W2T_ref[:, :], hid) + b2_ref[:, :]


def _tc_gat_body(flt_ref, lmT_ref, WlrT_ref, WeT_ref, att_ref, blr_ref, bias_ref,
                 W1T_ref, b1_ref, W2T_ref, b2_ref, outT_ref):
    flt = flt_ref[:, :]
    WlrT = WlrT_ref[:, :]
    blr = blr_ref[:, :]
    bias = bias_ref[:, :]
    att = att_ref[:, :]
    WeT = WeT_ref[:, :]

    xl, xr, pos = [], [], []
    for j in range(3):
        fj = flt[14 * j:14 * j + 14, :]
        xlr = _dot(WlrT, fj) + blr
        xl.append(xlr[0:16, :])
        xr.append(xlr[16:32, :])
        pos.append(fj[0:2, :])

    alphas = []
    for e in range(6):
        s, d = SRC[e], DST[e]
        cx = pos[d][0:1, :] - pos[s][0:1, :]
        cy = pos[d][1:2, :] - pos[s][1:2, :]
        dist = jnp.sqrt(cx * cx + cy * cy)
        eT = WeT[:, 0:1] * cx + WeT[:, 1:2] * cy + WeT[:, 2:3] * dist
        m = xl[s] + xr[d] + eT
        m = jnp.where(m > 0, m, 0.2 * m)
        alphas.append(jnp.sum(m * att, axis=0, keepdims=True))

    pool = jnp.zeros((16, TILE), _F32)
    for d in range(3):
        e1, s1, e2, s2 = IN_EDGES[d]
        a1, a2 = alphas[e1], alphas[e2]
        amax = jnp.maximum(a1, a2)
        x1 = jnp.exp(a1 - amax)
        x2 = jnp.exp(a2 - amax)
        den = x1 + x2 + 1e-16
        o = (x1 / den) * xl[s1] + (x2 / den) * xl[s2] + bias
        pool = pool + jnp.maximum(o, 0.0)

    AT = _obj_AT(WlrT[0:16, :])
    xlobj = _dot(AT, lmT_ref[:, :]) + blr[0:16, :] + bias
    objpool = 3.0 * jnp.maximum(xlobj, 0.0)

    h = jnp.concatenate([pool, objpool], axis=0)
    outT_ref[:, :] = _mlp(h, W1T_ref, b1_ref, W2T_ref, b2_ref)


def _tc_gat(flt, lmT, WlrT, WeT, att2, blr, bias2, W1T, b1, W2T, b2):
    grid = ((B - S) // TILE,)
    full = lambda shape: pl.BlockSpec(shape, lambda i: (0, 0))
    return pl.pallas_call(
        _tc_gat_body,
        grid=grid,
        in_specs=[
            pl.BlockSpec((42, TILE), lambda i: (0, i)),
            pl.BlockSpec((6, TILE), lambda i: (0, i)),
            full((32, 14)),
            full((16, 3)),
            full((16, 1)),
            full((32, 1)),
            full((16, 1)),
            full((128, 32)),
            full((128, 1)),
            full((32, 128)),
            full((32, 1)),
        ],
        out_specs=pl.BlockSpec((32, TILE), lambda i: (0, i)),
        out_shape=jax.ShapeDtypeStruct((32, B - S), _F32),
    )(flt, lmT, WlrT, WeT, att2, blr, bias2, W1T, b1, W2T, b2)


def _sc_head_body(pool_ref, lmT_ref, WlT_ref, blb_ref, W1T_ref, b1_ref,
                  W2T_ref, b2_ref, outT_ref):
    AT = _obj_AT(WlT_ref[:, :])
    xlobj = _dot(AT, lmT_ref[:, :]) + blb_ref[:, :]
    objpool = 3.0 * jnp.maximum(xlobj, 0.0)
    h = jnp.concatenate([pool_ref[:, :], objpool], axis=0)
    outT_ref[:, :] = _mlp(h, W1T_ref, b1_ref, W2T_ref, b2_ref)


TILE_HD = 2048


def _sc_head(pool, lmT, WlT, blb, W1T, b1, W2T, b2):
    grid = (S // TILE_HD,)
    full = lambda shape: pl.BlockSpec(shape, lambda i: (0, 0))
    return pl.pallas_call(
        _sc_head_body,
        grid=grid,
        in_specs=[
            pl.BlockSpec((16, TILE_HD), lambda i: (0, i)),
            pl.BlockSpec((6, TILE_HD), lambda i: (0, i)),
            full((16, 14)),
            full((16, 1)),
            full((128, 32)),
            full((128, 1)),
            full((32, 128)),
            full((32, 1)),
        ],
        out_specs=pl.BlockSpec((32, TILE_HD), lambda i: (0, i)),
        out_shape=jax.ShapeDtypeStruct((32, S), _F32),
    )(pool, lmT, WlT, blb, W1T, b1, W2T, b2)


@jax.jit
def _run(featT_sc, lmT_sc, featT_tc, lmT_tc, wtab, WlrT, WeT, att2, blr,
         bias2, blb, W1T, b1, W2T, b2):
    pool_sc = _sc_gat(featT_sc, wtab)
    out_tc = _tc_gat(featT_tc, lmT_tc, WlrT, WeT, att2, blr, bias2,
                     W1T, b1, W2T, b2)
    out_sc = _sc_head(pool_sc, lmT_sc, WlrT[0:16, :], blb, W1T, b1, W2T, b2)
    return jnp.concatenate([out_sc, out_tc], axis=1).T


def kernel(agent_pos, landmark_pos, agent_vel, other_pos, relative_landmark_pos,
           W_l, b_l, W_r, b_r, W_e, att, bias, W1, b1, W2, b2):
    b = agent_pos.shape[0]
    feat_sc = jnp.concatenate(
        [agent_pos[:S], agent_vel[:S], relative_landmark_pos[:S],
         other_pos[:S]], axis=2)
    feat_tc = jnp.concatenate(
        [agent_pos[S:], agent_vel[S:], relative_landmark_pos[S:],
         other_pos[S:]], axis=2)
    featT_sc = feat_sc.reshape(S, 42).T
    featT_tc = feat_tc.reshape(b - S, 42).T
    lmT_sc = landmark_pos[:S].reshape(S, 6).T
    lmT_tc = landmark_pos[S:].reshape(b - S, 6).T
    flat = jnp.concatenate([
        W_l.reshape(-1), W_r.reshape(-1), W_e.reshape(-1),
        att, b_l, b_r, bias])
    wtab = jnp.broadcast_to(flat[:, None], (_WROWS, 16))
    WlrT = jnp.concatenate([W_l.T, W_r.T], axis=0)
    blr = jnp.concatenate([b_l, b_r])[:, None]
    return _run(featT_sc, lmT_sc, featT_tc, lmT_tc, wtab, WlrT, W_e.T,
                att[:, None], blr, bias[:, None], (b_l + bias)[:, None],
                W1.T, b1[:, None], W2.T, b2[:, None])
